# Initial kernel scaffold; baseline (speedup 1.0000x reference)
import jax
import jax.numpy as jnp
from jax.experimental import pallas as pl


def kernel(inputs, pos_table):
    raise NotImplementedError("write your pallas kernel here")


# speedup vs baseline: 1.5011x; 1.5011x over previous
import jax
import jax.numpy as jnp
from jax.experimental import pallas as pl
from jax.experimental.pallas import tpu as pltpu

_BS = 512


def _add_body(in_ref, pos_ref, out_ref):
    out_ref[...] = in_ref[...] + pos_ref[...]


def kernel(inputs, pos_table):
    inputs = inputs.astype(jnp.float32)
    B, S, D = inputs.shape
    n_s = S // _BS
    flat = inputs.reshape(B * S, D)

    out = pl.pallas_call(
        _add_body,
        grid=(n_s, B),
        in_specs=[
            pl.BlockSpec((_BS, D), lambda s, b: (b * n_s + s, 0)),
            pl.BlockSpec((_BS, D), lambda s, b: (s, 0)),
        ],
        out_specs=pl.BlockSpec((_BS, D), lambda s, b: (b * n_s + s, 0)),
        out_shape=jax.ShapeDtypeStruct((B * S, D), jnp.float32),
        compiler_params=pltpu.CompilerParams(
            dimension_semantics=("arbitrary", "arbitrary"),
        ),
    )(flat, pos_table)
    return out.reshape(B, S, D)


# speedup vs baseline: 1.6673x; 1.1107x over previous
import jax
import jax.numpy as jnp
from jax.experimental import pallas as pl
from jax.experimental.pallas import tpu as pltpu

_BS = 1024


def _add_body(in_ref, pos_ref, out_ref):
    out_ref[...] = in_ref[...] + pos_ref[...]


def kernel(inputs, pos_table):
    inputs = inputs.astype(jnp.float32)
    B, S, D = inputs.shape
    n_s = S // _BS
    flat = inputs.reshape(B * S, D)

    out = pl.pallas_call(
        _add_body,
        grid=(n_s, B),
        in_specs=[
            pl.BlockSpec((_BS, D), lambda s, b: (b * n_s + s, 0)),
            pl.BlockSpec((_BS, D), lambda s, b: (s, 0)),
        ],
        out_specs=pl.BlockSpec((_BS, D), lambda s, b: (b * n_s + s, 0)),
        out_shape=jax.ShapeDtypeStruct((B * S, D), jnp.float32),
        compiler_params=pltpu.CompilerParams(
            dimension_semantics=("arbitrary", "arbitrary"),
        ),
    )(flat, pos_table)
    return out.reshape(B, S, D)


# speedup vs baseline: 1.7408x; 1.0441x over previous
import jax
import jax.numpy as jnp
from jax.experimental import pallas as pl
from jax.experimental.pallas import tpu as pltpu

_BS = 2048


def _add_body(in_ref, pos_ref, out_ref):
    out_ref[...] = in_ref[...] + pos_ref[...]


def kernel(inputs, pos_table):
    inputs = inputs.astype(jnp.float32)
    B, S, D = inputs.shape
    n_s = S // _BS
    flat = inputs.reshape(B * S, D)

    out = pl.pallas_call(
        _add_body,
        grid=(n_s, B),
        in_specs=[
            pl.BlockSpec((_BS, D), lambda s, b: (b * n_s + s, 0)),
            pl.BlockSpec((_BS, D), lambda s, b: (s, 0)),
        ],
        out_specs=pl.BlockSpec((_BS, D), lambda s, b: (b * n_s + s, 0)),
        out_shape=jax.ShapeDtypeStruct((B * S, D), jnp.float32),
        compiler_params=pltpu.CompilerParams(
            dimension_semantics=("arbitrary", "arbitrary"),
        ),
    )(flat, pos_table)
    return out.reshape(B, S, D)
